# Initial kernel scaffold; baseline (speedup 1.0000x reference)
import jax
import jax.numpy as jnp
from jax.experimental import pallas as pl


def kernel(x, edge_index, batch, W1, b1, W2, b2, candidates):
    raise NotImplementedError("write your pallas kernel here")


# speedup vs baseline: 18.3979x; 18.3979x over previous
import functools

import jax
import jax.numpy as jnp
from jax import lax
from jax.experimental import pallas as pl
from jax.experimental.pallas import tpu as pltpu
from jax.experimental.pallas import tpu_sc as plsc

N = 10000
E = 320000
D = 128
H = 128
O = 64
G = 64
C = 64

NC = 2
NS = 16
NW = NC * NS
EPW = E // NW
CHUNK = 80
NCHUNK = EPW // CHUNK
NP = 10240
RPS = NP // NS
ZROWS = 128


def _make_deg():
    mesh = plsc.VectorSubcoreMesh(core_axis_name="c", subcore_axis_name="s")

    @functools.partial(
        pl.kernel,
        out_type=jax.ShapeDtypeStruct((NC, NP, H), jnp.float32),
        mesh=mesh,
        scratch_types=[
            pltpu.VMEM((NCHUNK, CHUNK), jnp.int32),
            pltpu.VMEM((CHUNK, H), jnp.float32),
            pltpu.VMEM_SHARED((NP, H), jnp.float32),
        ],
    )
    def deg_kernel(dst_hbm, out_hbm, dst_ids, buf, acc):
        cid = lax.axis_index("c")
        sid = lax.axis_index("s")
        wid = cid * NS + sid

        def fill_z(i, carry):
            for c in range(H // 16):
                buf[i, pl.ds(c * 16, 16)] = jnp.zeros((16,), jnp.float32)
            return carry

        lax.fori_loop(0, CHUNK, fill_z, 0)
        base = sid * RPS
        for k in range(RPS // CHUNK):
            pltpu.sync_copy(buf, acc.at[pl.ds(base + k * CHUNK, CHUNK)])
        pltpu.sync_copy(dst_hbm.at[wid], dst_ids)

        def fill_one(i, carry):
            for c in range(H // 16):
                buf[i, pl.ds(c * 16, 16)] = jnp.ones((16,), jnp.float32)
            return carry

        lax.fori_loop(0, CHUNK, fill_one, 0)
        plsc.subcore_barrier()

        def body(j, carry):
            pltpu.sync_copy(buf, acc.at[dst_ids.at[j]], add=True)
            return carry

        lax.fori_loop(0, NCHUNK, body, 0)
        plsc.subcore_barrier()

        for k in range(RPS // CHUNK):
            pltpu.sync_copy(acc.at[pl.ds(base + k * CHUNK, CHUNK)], buf)
            pltpu.sync_copy(buf, out_hbm.at[cid, pl.ds(base + k * CHUNK, CHUNK)])

    return deg_kernel


def _make_scatter():
    mesh = plsc.VectorSubcoreMesh(core_axis_name="c", subcore_axis_name="s")

    @functools.partial(
        pl.kernel,
        out_type=jax.ShapeDtypeStruct((NC, NP, H), jnp.float32),
        mesh=mesh,
        scratch_types=[
            pltpu.VMEM((NCHUNK, CHUNK), jnp.int32),
            pltpu.VMEM((NCHUNK, CHUNK), jnp.int32),
            pltpu.VMEM((CHUNK, H), jnp.float32),
            pltpu.VMEM_SHARED((NP, H), jnp.float32),
            pltpu.SemaphoreType.DMA,
        ],
    )
    def scat_kernel(vals_hbm, src_hbm, dst_hbm, out_hbm,
                    src_ids, dst_ids, rows, acc, sem):
        cid = lax.axis_index("c")
        sid = lax.axis_index("s")
        wid = cid * NS + sid

        def fill_z(i, carry):
            for c in range(H // 16):
                rows[i, pl.ds(c * 16, 16)] = jnp.zeros((16,), jnp.float32)
            return carry

        lax.fori_loop(0, CHUNK, fill_z, 0)

        base = sid * RPS
        for k in range(RPS // CHUNK):
            pltpu.sync_copy(rows, acc.at[pl.ds(base + k * CHUNK, CHUNK)])
        pltpu.sync_copy(src_hbm.at[wid], src_ids)
        pltpu.sync_copy(dst_hbm.at[wid], dst_ids)
        plsc.subcore_barrier()

        def body(j, carry):
            pltpu.async_copy(vals_hbm.at[src_ids.at[j]], rows, sem).wait()
            pltpu.sync_copy(rows, acc.at[dst_ids.at[j]], add=True)
            return carry

        lax.fori_loop(0, NCHUNK, body, 0)
        plsc.subcore_barrier()

        for k in range(RPS // CHUNK):
            pltpu.sync_copy(acc.at[pl.ds(base + k * CHUNK, CHUNK)], rows)
            pltpu.sync_copy(rows, out_hbm.at[cid, pl.ds(base + k * CHUNK, CHUNK)])

    return scat_kernel


_deg = _make_deg()
_scatter = _make_scatter()


def _mm_body(x_ref, w_ref, o_ref):
    o_ref[...] = jnp.dot(x_ref[...], w_ref[...],
                         preferred_element_type=jnp.float32)


def _matmul(x, w):
    n, d = x.shape
    h = w.shape[1]
    blk = 2048
    return pl.pallas_call(
        _mm_body,
        grid=(n // blk,),
        in_specs=[pl.BlockSpec((blk, d), lambda i: (i, 0)),
                  pl.BlockSpec((d, h), lambda i: (0, 0))],
        out_specs=pl.BlockSpec((blk, h), lambda i: (i, 0)),
        out_shape=jax.ShapeDtypeStruct((n, h), jnp.float32),
    )(x, w)


def _scale_body(y_ref, degp_ref, g_ref, dinv_ref):
    deg = degp_ref[0, :, 0:1] + degp_ref[1, :, 0:1] + 1.0
    dinv = lax.rsqrt(jnp.maximum(deg, 1.0))
    dinv_ref[...] = dinv
    g_ref[...] = y_ref[...] * dinv


def _scale(y, degp):
    blk = 2048
    return pl.pallas_call(
        _scale_body,
        grid=(NP // blk,),
        in_specs=[pl.BlockSpec((blk, H), lambda i: (i, 0)),
                  pl.BlockSpec((NC, blk, H), lambda i: (0, i, 0))],
        out_specs=[pl.BlockSpec((blk, H), lambda i: (i, 0)),
                   pl.BlockSpec((blk, 1), lambda i: (i, 0))],
        out_shape=[jax.ShapeDtypeStruct((NP, H), jnp.float32),
                   jax.ShapeDtypeStruct((NP, 1), jnp.float32)],
    )(y, degp)


def _fuse2_body(agg_ref, y1_ref, dinv_ref, b1_ref, w2_ref, g2b_ref, g2f_ref):
    dinv = dinv_ref[...]
    a = agg_ref[0] + agg_ref[1] + y1_ref[...] * dinv
    h = jnp.maximum(a * dinv + b1_ref[...], 0.0)
    g2 = jnp.dot(h, w2_ref[...], preferred_element_type=jnp.float32) * dinv
    g2f_ref[...] = g2
    g2b_ref[...] = jnp.concatenate([g2, jnp.zeros_like(g2)], axis=1)


def _fuse2(agg1, y1, dinv, b1, W2):
    blk = 2048
    return pl.pallas_call(
        _fuse2_body,
        grid=(NP // blk,),
        in_specs=[pl.BlockSpec((NC, blk, H), lambda i: (0, i, 0)),
                  pl.BlockSpec((blk, H), lambda i: (i, 0)),
                  pl.BlockSpec((blk, 1), lambda i: (i, 0)),
                  pl.BlockSpec((1, H), lambda i: (0, 0)),
                  pl.BlockSpec((H, O), lambda i: (0, 0))],
        out_specs=[pl.BlockSpec((blk, H), lambda i: (i, 0)),
                   pl.BlockSpec((blk, O), lambda i: (i, 0))],
        out_shape=[jax.ShapeDtypeStruct((NP, H), jnp.float32),
                   jax.ShapeDtypeStruct((NP, O), jnp.float32)],
    )(agg1, y1, dinv, b1, W2)


def _final_body(agg_ref, g2f_ref, dinv_ref, b2_ref, batch_ref, cand_ref,
                out_ref):
    emb = ((agg_ref[0, :, :O] + agg_ref[1, :, :O] + g2f_ref[...])
           * dinv_ref[...] + b2_ref[...])
    gids = lax.broadcasted_iota(jnp.int32, (G, NP), 0)
    mask = (gids == batch_ref[...]).astype(jnp.float32)
    sums = jnp.dot(mask, emb, preferred_element_type=jnp.float32)
    counts = jnp.sum(mask, axis=1, keepdims=True)
    gr = sums / jnp.maximum(counts, 1.0)
    cand = cand_ref[...]
    gr2 = jnp.sum(gr * gr, axis=1, keepdims=True)
    cn2 = jnp.sum(cand * cand, axis=1)[None, :]
    cross = lax.dot_general(gr, cand, (((1,), (1,)), ((), ())),
                            preferred_element_type=jnp.float32)
    d2 = jnp.maximum(gr2 + cn2 - 2.0 * cross, 0.0)
    out_ref[...] = jnp.sqrt(d2)


def _final(agg2, g2f, dinv, b2, batch2d, cand):
    return pl.pallas_call(
        _final_body,
        grid=(1,),
        in_specs=[pl.BlockSpec((NC, NP, H), lambda i: (0, 0, 0)),
                  pl.BlockSpec((NP, O), lambda i: (0, 0)),
                  pl.BlockSpec((NP, 1), lambda i: (0, 0)),
                  pl.BlockSpec((1, O), lambda i: (0, 0)),
                  pl.BlockSpec((1, NP), lambda i: (0, 0)),
                  pl.BlockSpec((C, O), lambda i: (0, 0))],
        out_specs=pl.BlockSpec((G, C), lambda i: (0, 0)),
        out_shape=jax.ShapeDtypeStruct((G, C), jnp.float32),
    )(agg2, g2f, dinv, b2, batch2d, cand)


def kernel(x, edge_index, batch, W1, b1, W2, b2, candidates):
    src3 = edge_index[0].reshape(NW, NCHUNK, CHUNK)
    dst3 = edge_index[1].reshape(NW, NCHUNK, CHUNK)
    x_p = jnp.pad(x, ((0, NP - N), (0, 0)))
    batch_p = jnp.pad(batch, (0, NP - N), constant_values=G)
    degp = _deg(dst3)
    y1 = _matmul(x_p, W1)
    g1b, dinv = _scale(y1, degp)
    agg1 = _scatter(g1b, src3, dst3)
    g2b, g2f = _fuse2(agg1, y1, dinv, b1.reshape(1, H), W2)
    agg2 = _scatter(g2b, src3, dst3)
    sims = _final(agg2, g2f, dinv, b2.reshape(1, O),
                  batch_p.reshape(1, NP), candidates)
    return sims[:, :, None]


# speedup vs baseline: 21.7725x; 1.1834x over previous
import functools

import jax
import jax.numpy as jnp
from jax import lax
from jax.experimental import pallas as pl
from jax.experimental.pallas import tpu as pltpu
from jax.experimental.pallas import tpu_sc as plsc

N = 10000
E = 320000
D = 128
H = 128
O = 64
G = 64
C = 64

NC = 2
NS = 16
NW = NC * NS
EPW = E // NW
CHUNK = 40
NCHUNK = EPW // CHUNK
NP = 10240
RPS = NP // NS
ZROWS = 128


def _make_deg():
    mesh = plsc.VectorSubcoreMesh(core_axis_name="c", subcore_axis_name="s")

    @functools.partial(
        pl.kernel,
        out_type=jax.ShapeDtypeStruct((NC, NP, H), jnp.float32),
        mesh=mesh,
        scratch_types=[
            pltpu.VMEM((EPW,), jnp.int32),
            pltpu.VMEM((CHUNK, H), jnp.float32),
            pltpu.VMEM_SHARED((NP, H), jnp.float32),
        ],
    )
    def deg_kernel(dst_hbm, out_hbm, dst_ids, buf, acc):
        cid = lax.axis_index("c")
        sid = lax.axis_index("s")
        wid = cid * NS + sid

        def fill_z(i, carry):
            for c in range(H // 16):
                buf[i, pl.ds(c * 16, 16)] = jnp.zeros((16,), jnp.float32)
            return carry

        lax.fori_loop(0, CHUNK, fill_z, 0)
        base = sid * RPS
        for k in range(RPS // CHUNK):
            pltpu.sync_copy(buf, acc.at[pl.ds(base + k * CHUNK, CHUNK)])
        pltpu.sync_copy(dst_hbm.at[pl.ds(wid * EPW, EPW)], dst_ids)

        def fill_one(i, carry):
            for c in range(H // 16):
                buf[i, pl.ds(c * 16, 16)] = jnp.ones((16,), jnp.float32)
            return carry

        lax.fori_loop(0, CHUNK, fill_one, 0)
        plsc.subcore_barrier()

        def body(j, carry):
            pltpu.sync_copy(buf, acc.at[dst_ids.at[pl.ds(j * CHUNK, CHUNK)]],
                            add=True)
            return carry

        lax.fori_loop(0, NCHUNK, body, 0)
        plsc.subcore_barrier()

        for k in range(RPS // CHUNK):
            pltpu.sync_copy(acc.at[pl.ds(base + k * CHUNK, CHUNK)], buf)
            pltpu.sync_copy(buf, out_hbm.at[cid, pl.ds(base + k * CHUNK, CHUNK)])

    return deg_kernel


def _make_scatter():
    mesh = plsc.VectorSubcoreMesh(core_axis_name="c", subcore_axis_name="s")

    @functools.partial(
        pl.kernel,
        out_type=jax.ShapeDtypeStruct((NC, NP, H), jnp.float32),
        mesh=mesh,
        scratch_types=[
            pltpu.VMEM((EPW,), jnp.int32),
            pltpu.VMEM((EPW,), jnp.int32),
            pltpu.VMEM((CHUNK, H), jnp.float32),
            pltpu.VMEM((CHUNK, H), jnp.float32),
            pltpu.VMEM_SHARED((NP, H), jnp.float32),
            pltpu.SemaphoreType.DMA,
            pltpu.SemaphoreType.DMA,
        ],
    )
    def scat_kernel(vals_hbm, src_hbm, dst_hbm, out_hbm,
                    src_ids, dst_ids, rows0, rows1, acc, sem0, sem1):
        cid = lax.axis_index("c")
        sid = lax.axis_index("s")
        wid = cid * NS + sid

        def fill_z(i, carry):
            for c in range(H // 16):
                rows0[i, pl.ds(c * 16, 16)] = jnp.zeros((16,), jnp.float32)
            return carry

        lax.fori_loop(0, CHUNK, fill_z, 0)

        base = sid * RPS
        for k in range(RPS // CHUNK):
            pltpu.sync_copy(rows0, acc.at[pl.ds(base + k * CHUNK, CHUNK)])
        pltpu.sync_copy(src_hbm.at[pl.ds(wid * EPW, EPW)], src_ids)
        pltpu.sync_copy(dst_hbm.at[pl.ds(wid * EPW, EPW)], dst_ids)
        plsc.subcore_barrier()

        pltpu.async_copy(vals_hbm.at[src_ids.at[pl.ds(0, CHUNK)]], rows0, sem0)

        def body(i, carry):
            e0 = 2 * i * CHUNK
            e1 = e0 + CHUNK
            e2 = jnp.minimum(e1 + CHUNK, EPW - CHUNK)
            pltpu.async_copy(vals_hbm.at[src_ids.at[pl.ds(e1, CHUNK)]],
                             rows1, sem1)
            pltpu.make_async_copy(vals_hbm.at[src_ids.at[pl.ds(e0, CHUNK)]],
                                  rows0, sem0).wait()
            pltpu.sync_copy(rows0, acc.at[dst_ids.at[pl.ds(e0, CHUNK)]],
                            add=True)
            pltpu.async_copy(vals_hbm.at[src_ids.at[pl.ds(e2, CHUNK)]],
                             rows0, sem0)
            pltpu.make_async_copy(vals_hbm.at[src_ids.at[pl.ds(e1, CHUNK)]],
                                  rows1, sem1).wait()
            pltpu.sync_copy(rows1, acc.at[dst_ids.at[pl.ds(e1, CHUNK)]],
                            add=True)
            return carry

        lax.fori_loop(0, NCHUNK // 2, body, 0)
        pltpu.make_async_copy(vals_hbm.at[src_ids.at[pl.ds(0, CHUNK)]],
                              rows0, sem0).wait()
        plsc.subcore_barrier()

        for k in range(RPS // CHUNK):
            pltpu.sync_copy(acc.at[pl.ds(base + k * CHUNK, CHUNK)], rows0)
            pltpu.sync_copy(rows0, out_hbm.at[cid, pl.ds(base + k * CHUNK, CHUNK)])

    return scat_kernel


_deg = _make_deg()
_scatter = _make_scatter()


def _mm_body(x_ref, w_ref, o_ref):
    o_ref[...] = jnp.dot(x_ref[...], w_ref[...],
                         preferred_element_type=jnp.float32)


def _matmul(x, w):
    n, d = x.shape
    h = w.shape[1]
    blk = 2048
    return pl.pallas_call(
        _mm_body,
        grid=(n // blk,),
        in_specs=[pl.BlockSpec((blk, d), lambda i: (i, 0)),
                  pl.BlockSpec((d, h), lambda i: (0, 0))],
        out_specs=pl.BlockSpec((blk, h), lambda i: (i, 0)),
        out_shape=jax.ShapeDtypeStruct((n, h), jnp.float32),
    )(x, w)


def _scale_body(y_ref, degp_ref, g_ref, dinv_ref):
    deg = degp_ref[0, :, 0:1] + degp_ref[1, :, 0:1] + 1.0
    dinv = lax.rsqrt(jnp.maximum(deg, 1.0))
    dinv_ref[...] = dinv
    g_ref[...] = y_ref[...] * dinv


def _scale(y, degp):
    blk = 2048
    return pl.pallas_call(
        _scale_body,
        grid=(NP // blk,),
        in_specs=[pl.BlockSpec((blk, H), lambda i: (i, 0)),
                  pl.BlockSpec((NC, blk, H), lambda i: (0, i, 0))],
        out_specs=[pl.BlockSpec((blk, H), lambda i: (i, 0)),
                   pl.BlockSpec((blk, 1), lambda i: (i, 0))],
        out_shape=[jax.ShapeDtypeStruct((NP, H), jnp.float32),
                   jax.ShapeDtypeStruct((NP, 1), jnp.float32)],
    )(y, degp)


def _fuse2_body(agg_ref, y1_ref, dinv_ref, b1_ref, w2_ref, g2b_ref, g2f_ref):
    dinv = dinv_ref[...]
    a = agg_ref[0] + agg_ref[1] + y1_ref[...] * dinv
    h = jnp.maximum(a * dinv + b1_ref[...], 0.0)
    g2 = jnp.dot(h, w2_ref[...], preferred_element_type=jnp.float32) * dinv
    g2f_ref[...] = g2
    g2b_ref[...] = jnp.concatenate([g2, jnp.zeros_like(g2)], axis=1)


def _fuse2(agg1, y1, dinv, b1, W2):
    blk = 2048
    return pl.pallas_call(
        _fuse2_body,
        grid=(NP // blk,),
        in_specs=[pl.BlockSpec((NC, blk, H), lambda i: (0, i, 0)),
                  pl.BlockSpec((blk, H), lambda i: (i, 0)),
                  pl.BlockSpec((blk, 1), lambda i: (i, 0)),
                  pl.BlockSpec((1, H), lambda i: (0, 0)),
                  pl.BlockSpec((H, O), lambda i: (0, 0))],
        out_specs=[pl.BlockSpec((blk, H), lambda i: (i, 0)),
                   pl.BlockSpec((blk, O), lambda i: (i, 0))],
        out_shape=[jax.ShapeDtypeStruct((NP, H), jnp.float32),
                   jax.ShapeDtypeStruct((NP, O), jnp.float32)],
    )(agg1, y1, dinv, b1, W2)


def _final_body(agg_ref, g2f_ref, dinv_ref, b2_ref, batch_ref, cand_ref,
                out_ref):
    emb = ((agg_ref[0, :, :O] + agg_ref[1, :, :O] + g2f_ref[...])
           * dinv_ref[...] + b2_ref[...])
    gids = lax.broadcasted_iota(jnp.int32, (G, NP), 0)
    mask = (gids == batch_ref[...]).astype(jnp.float32)
    sums = jnp.dot(mask, emb, preferred_element_type=jnp.float32)
    counts = jnp.sum(mask, axis=1, keepdims=True)
    gr = sums / jnp.maximum(counts, 1.0)
    cand = cand_ref[...]
    gr2 = jnp.sum(gr * gr, axis=1, keepdims=True)
    cn2 = jnp.sum(cand * cand, axis=1)[None, :]
    cross = lax.dot_general(gr, cand, (((1,), (1,)), ((), ())),
                            preferred_element_type=jnp.float32)
    d2 = jnp.maximum(gr2 + cn2 - 2.0 * cross, 0.0)
    out_ref[...] = jnp.sqrt(d2)


def _final(agg2, g2f, dinv, b2, batch2d, cand):
    return pl.pallas_call(
        _final_body,
        grid=(1,),
        in_specs=[pl.BlockSpec((NC, NP, H), lambda i: (0, 0, 0)),
                  pl.BlockSpec((NP, O), lambda i: (0, 0)),
                  pl.BlockSpec((NP, 1), lambda i: (0, 0)),
                  pl.BlockSpec((1, O), lambda i: (0, 0)),
                  pl.BlockSpec((1, NP), lambda i: (0, 0)),
                  pl.BlockSpec((C, O), lambda i: (0, 0))],
        out_specs=pl.BlockSpec((G, C), lambda i: (0, 0)),
        out_shape=jax.ShapeDtypeStruct((G, C), jnp.float32),
    )(agg2, g2f, dinv, b2, batch2d, cand)


def kernel(x, edge_index, batch, W1, b1, W2, b2, candidates):
    src3 = edge_index[0]
    dst3 = edge_index[1]
    x_p = jnp.pad(x, ((0, NP - N), (0, 0)))
    batch_p = jnp.pad(batch, (0, NP - N), constant_values=G)
    degp = _deg(dst3)
    y1 = _matmul(x_p, W1)
    g1b, dinv = _scale(y1, degp)
    agg1 = _scatter(g1b, src3, dst3)
    g2b, g2f = _fuse2(agg1, y1, dinv, b1.reshape(1, H), W2)
    agg2 = _scatter(g2b, src3, dst3)
    sims = _final(agg2, g2f, dinv, b2.reshape(1, O),
                  batch_p.reshape(1, NP), candidates)
    return sims[:, :, None]
